# baseline (speedup 1.0000x reference)
import jax
import jax.numpy as jnp
from jax import lax
from jax.experimental import pallas as pl
from jax.experimental.pallas import tpu as pltpu
from jax.experimental.pallas import tpu_sc as plsc

BATCH = 16384
EMB_DIM = 64
_INFO = plsc.get_sparse_core_info()
_NC, _NS, _L = _INFO.num_cores, _INFO.num_subcores, _INFO.num_lanes
_NW = _NC * _NS
_BPW = BATCH // _NW
_NCHUNK = _BPW // _L
_DEPTH = 3


def _body(uid_hbm, bid_hbm, utab_hbm, btab_hbm, out_hbm,
          uid_v, bid_v, ub0, ub1, ub2, bb0, bb1, bb2, out_v,
          us0, us1, us2, bs0, bs1, bs2):
    wid = lax.axis_index("s") * _NC + lax.axis_index("c")
    base = wid * _BPW

    pltpu.sync_copy(uid_hbm.at[pl.ds(base, _BPW)], uid_v)
    pltpu.sync_copy(bid_hbm.at[pl.ds(base, _BPW)], bid_v)

    ubufs, bbufs = (ub0, ub1, ub2), (bb0, bb1, bb2)
    usems, bsems = (us0, us1, us2), (bs0, bs1, bs2)
    lane = lax.iota(jnp.int32, _L)

    def ids(c):
        return uid_v[pl.ds(c * _L, _L)], bid_v[pl.ds(c * _L, _L)]

    def compute(c, k):
        uvec, bvec = ids(c)
        uhalf = (uvec & 1) << 6
        bhalf = (bvec & 1) << 6

        def step(d, acc):
            col = lax.bitwise_and(d + lane, EMB_DIM - 1)
            u = plsc.load_gather(ubufs[k], [lane, uhalf + col])
            b = plsc.load_gather(bbufs[k], [lane, bhalf + col])
            return acc + u * b

        acc = lax.fori_loop(0, EMB_DIM, step, jnp.zeros((_L,), jnp.float32))
        out_v[pl.ds(c * _L, _L)] = acc

    def fire(c, k):
        uvec, bvec = ids(c)
        return (
            pltpu.async_copy(utab_hbm.at[uvec >> 1], ubufs[k], usems[k]),
            pltpu.async_copy(btab_hbm.at[bvec >> 1], bbufs[k], bsems[k]),
        )

    def stage(t, nfire):
        c0 = t * _DEPTH
        fired = [fire(c0 + s, s) for s in range(nfire)]
        for s in range(nfire):
            for cp in fired[s]:
                cp.wait()
            compute(c0 + s, s)
        return 0

    lax.fori_loop(0, _NCHUNK // _DEPTH, lambda t, x: stage(t, _DEPTH), 0)
    if _NCHUNK % _DEPTH:
        stage(_NCHUNK // _DEPTH, _NCHUNK % _DEPTH)

    pltpu.sync_copy(out_v, out_hbm.at[pl.ds(base, _BPW)])


@jax.jit
def _run(uid, bid, utab, btab):
    mesh = plsc.VectorSubcoreMesh(core_axis_name="c", subcore_axis_name="s")
    return pl.kernel(
        _body,
        mesh=mesh,
        compiler_params=pltpu.CompilerParams(needs_layout_passes=False),
        out_type=jax.ShapeDtypeStruct((BATCH,), jnp.float32),
        scratch_types=[
            pltpu.VMEM((_BPW,), jnp.int32),
            pltpu.VMEM((_BPW,), jnp.int32),
            pltpu.VMEM((_L, 2 * EMB_DIM), jnp.float32),
            pltpu.VMEM((_L, 2 * EMB_DIM), jnp.float32),
            pltpu.VMEM((_L, 2 * EMB_DIM), jnp.float32),
            pltpu.VMEM((_L, 2 * EMB_DIM), jnp.float32),
            pltpu.VMEM((_L, 2 * EMB_DIM), jnp.float32),
            pltpu.VMEM((_L, 2 * EMB_DIM), jnp.float32),
            pltpu.VMEM((_BPW,), jnp.float32),
            pltpu.SemaphoreType.DMA,
            pltpu.SemaphoreType.DMA,
            pltpu.SemaphoreType.DMA,
            pltpu.SemaphoreType.DMA,
            pltpu.SemaphoreType.DMA,
            pltpu.SemaphoreType.DMA,
        ],
    )(uid, bid, utab, btab)


def kernel(user_ids, banner_ids, user_table, banner_table):
    utab2 = user_table.reshape(-1, 2 * EMB_DIM)
    btab2 = banner_table.reshape(-1, 2 * EMB_DIM)
    return _run(user_ids.astype(jnp.int32), banner_ids.astype(jnp.int32),
                utab2, btab2)
